# Initial kernel scaffold; baseline (speedup 1.0000x reference)
import jax
import jax.numpy as jnp
from jax.experimental import pallas as pl


def kernel(x, edge_index, W, b, W_ih, W_hh, b_ih, b_hh):
    raise NotImplementedError("write your pallas kernel here")


# speedup vs baseline: 7.6734x; 7.6734x over previous
import functools

import jax
import jax.numpy as jnp
from jax import lax
from jax.experimental import pallas as pl
from jax.experimental.pallas import tpu as pltpu
from jax.experimental.pallas import tpu_sc as plsc

N = 10000
D = 128
E = 320000

NC = 2
NS = 16
NW = NC * NS

K = 128
CHUNKS = -(-E // (NW * K))
EPW = CHUNKS * K
EPAD = EPW * NW

ROWS_PER_TILE = 640
NSH = ROWS_PER_TILE * NS


def _linear_body(x_ref, w_ref, b_ref, y_ref):
    y_ref[...] = lax.dot_general(
        x_ref[...], w_ref[...], (((1,), (1,)), ((), ())),
        preferred_element_type=jnp.float32) + b_ref[...]


def _linear(x, w, b2):
    blk = 1000
    return pl.pallas_call(
        _linear_body,
        grid=(N // blk,),
        in_specs=[
            pl.BlockSpec((blk, D), lambda i: (i, 0)),
            pl.BlockSpec((D, D), lambda i: (0, 0)),
            pl.BlockSpec((1, D), lambda i: (0, 0)),
        ],
        out_specs=pl.BlockSpec((blk, D), lambda i: (i, 0)),
        out_shape=jax.ShapeDtypeStruct((N, D), jnp.float32),
    )(x, w, b2)


def _sc_body(y_hbm, src_hbm, dst_hbm, zrow_hbm, zcnt_hbm, ones_hbm,
             agg_out, cnt_out,
             src_v, dst_v, rows_v, ones_v, agg_sh, cnt_sh, sem):
    c = lax.axis_index("c")
    s = lax.axis_index("s")
    wid = s * NC + c

    pltpu.sync_copy(zrow_hbm, agg_sh.at[pl.ds(s * ROWS_PER_TILE, ROWS_PER_TILE)])
    pltpu.sync_copy(zcnt_hbm, cnt_sh.at[pl.ds(s * ROWS_PER_TILE, ROWS_PER_TILE)])
    pltpu.sync_copy(src_hbm.at[wid], src_v)
    pltpu.sync_copy(dst_hbm.at[wid], dst_v)
    pltpu.sync_copy(ones_hbm, ones_v)
    plsc.subcore_barrier()

    def chunk(j, carry):
        pltpu.async_copy(y_hbm.at[src_v.at[j]], rows_v, sem).wait()
        pltpu.sync_copy(rows_v, agg_sh.at[dst_v.at[j]], add=True)
        pltpu.sync_copy(ones_v, cnt_sh.at[dst_v.at[j]], add=True)
        return carry

    lax.fori_loop(0, CHUNKS, chunk, 0)
    plsc.subcore_barrier()

    pltpu.sync_copy(agg_sh.at[pl.ds(s * ROWS_PER_TILE, ROWS_PER_TILE)],
                    agg_out.at[c, pl.ds(s * ROWS_PER_TILE, ROWS_PER_TILE)])
    pltpu.sync_copy(cnt_sh.at[pl.ds(s * ROWS_PER_TILE, ROWS_PER_TILE)],
                    cnt_out.at[c, pl.ds(s * ROWS_PER_TILE, ROWS_PER_TILE)])


def _sc_scatter(y, src_r, dst_r, zrow, zcnt, ones_k):
    mesh = plsc.VectorSubcoreMesh(core_axis_name="c", subcore_axis_name="s")
    fn = pl.kernel(
        _sc_body,
        out_type=(
            jax.ShapeDtypeStruct((NC, NSH, D), jnp.float32),
            jax.ShapeDtypeStruct((NC, NSH, 16), jnp.float32),
        ),
        mesh=mesh,
        scratch_types=[
            pltpu.VMEM((CHUNKS, K), jnp.int32),
            pltpu.VMEM((CHUNKS, K), jnp.int32),
            pltpu.VMEM((K, D), jnp.float32),
            pltpu.VMEM((K, 16), jnp.float32),
            pltpu.VMEM_SHARED((NSH, D), jnp.float32),
            pltpu.VMEM_SHARED((NSH, 16), jnp.float32),
            pltpu.SemaphoreType.DMA,
        ],
        compiler_params=pltpu.CompilerParams(use_tc_tiling_on_sc=False),
    )
    return fn(y, src_r, dst_r, zrow, zcnt, ones_k)


def _gru_body(y_ref, part_ref, cnt_ref, wih_ref, whh_ref, bih_ref, bhh_ref,
              out_ref):
    y = y_ref[...]
    agg = part_ref[0] + part_ref[1] + y
    cnt = cnt_ref[0, :, 0:1] + cnt_ref[1, :, 0:1] + 1.0
    h = agg / cnt

    w_ih = wih_ref[...]
    w_hh = whh_ref[...]
    b_ih = bih_ref[...]
    b_hh = bhh_ref[...]

    dn = (((1,), (1,)), ((), ()))
    gi1 = lax.dot_general(h, w_ih, dn, preferred_element_type=jnp.float32) + b_ih
    z1 = jax.nn.sigmoid(gi1[:, 128:256] + b_hh[:, 128:256])
    r1 = jax.nn.sigmoid(gi1[:, 0:128] + b_hh[:, 0:128])
    n1 = jnp.tanh(gi1[:, 256:384] + r1 * b_hh[:, 256:384])
    h1 = (1.0 - z1) * n1
    gi2 = lax.dot_general(y, w_ih, dn, preferred_element_type=jnp.float32) + b_ih
    gh2 = lax.dot_general(h1, w_hh, dn, preferred_element_type=jnp.float32) + b_hh
    r2 = jax.nn.sigmoid(gi2[:, 0:128] + gh2[:, 0:128])
    z2 = jax.nn.sigmoid(gi2[:, 128:256] + gh2[:, 128:256])
    n2 = jnp.tanh(gi2[:, 256:384] + r2 * gh2[:, 256:384])
    out_ref[...] = (1.0 - z2) * n2 + z2 * h1


def _mean_gru(y, part, cntp, w_ih, w_hh, b_ih2, b_hh2):
    blk = 1000
    return pl.pallas_call(
        _gru_body,
        grid=(N // blk,),
        in_specs=[
            pl.BlockSpec((blk, D), lambda i: (i, 0)),
            pl.BlockSpec((NC, blk, D), lambda i: (0, i, 0)),
            pl.BlockSpec((NC, blk, 16), lambda i: (0, i, 0)),
            pl.BlockSpec((3 * D, D), lambda i: (0, 0)),
            pl.BlockSpec((3 * D, D), lambda i: (0, 0)),
            pl.BlockSpec((1, 3 * D), lambda i: (0, 0)),
            pl.BlockSpec((1, 3 * D), lambda i: (0, 0)),
        ],
        out_specs=pl.BlockSpec((blk, D), lambda i: (i, 0)),
        out_shape=jax.ShapeDtypeStruct((N, D), jnp.float32),
    )(y, part, cntp, w_ih, w_hh, b_ih2, b_hh2)


def kernel(x, edge_index, W, b, W_ih, W_hh, b_ih, b_hh):
    src = edge_index[0].astype(jnp.int32)
    dst = edge_index[1].astype(jnp.int32)
    pad = EPAD - E
    src_r = jnp.concatenate([src, jnp.zeros((pad,), jnp.int32)]).reshape(NW, CHUNKS, K)
    dst_r = jnp.concatenate([dst, jnp.full((pad,), N, jnp.int32)]).reshape(NW, CHUNKS, K)

    zrow = jnp.zeros((ROWS_PER_TILE, D), jnp.float32)
    zcnt = jnp.zeros((ROWS_PER_TILE, 16), jnp.float32)
    ones_k = jnp.ones((K, 16), jnp.float32)

    y = _linear(x, W, b.reshape(1, D))
    part, cntp = _sc_scatter(y, src_r, dst_r, zrow, zcnt, ones_k)
    return _mean_gru(y, part, cntp, W_ih, W_hh,
                     b_ih.reshape(1, 3 * D), b_hh.reshape(1, 3 * D))
